# baseline (speedup 1.0000x reference)
import functools

import jax
import jax.numpy as jnp
from jax import lax
from jax.experimental import pallas as pl
from jax.experimental.pallas import tpu as pltpu
from jax.experimental.pallas import tpu_sc as plsc

B, Q, C, T = 64, 4096, 16, 512
BG = 15
W_CLASS, W_DIR, W_RAD = 1.0, 5.0, 2.0

N_ITEMS = B * T
NW = 32
N_PER_W = N_ITEMS // NW
LG = N_PER_W // 16
IDX_CHUNK = 128
N_CHUNKS = N_PER_W // IDX_CHUNK


_TC_ROWS = 512
_TC_COLS = 1024
_TC_GRID = (B * Q * C) // (_TC_ROWS * _TC_COLS)


def _bg_focal_body(x_ref, coeff_ref, out_ref):
    i = pl.program_id(0)

    @pl.when(i == 0)
    def _init():
        out_ref[...] = jnp.zeros((1, 1), jnp.float32)

    x = x_ref[...]
    lane = lax.broadcasted_iota(jnp.int32, (_TC_ROWS, _TC_COLS), 1)
    is_bg = (lane % C) == BG
    p = jax.nn.sigmoid(x)
    sp_pos = jnp.maximum(x, 0.0) + jnp.log1p(jnp.exp(-jnp.abs(x)))
    mod = jnp.where(is_bg, (1.0 - p) * (1.0 - p) * (sp_pos - x),
                    p * p * sp_pos)
    out_ref[...] += jnp.sum(mod * coeff_ref[...]).reshape(1, 1)


def _bg_focal_sum(class_logits, coeff_tile):
    x2 = class_logits.reshape(-1, _TC_COLS)
    out = pl.pallas_call(
        _bg_focal_body,
        grid=(_TC_GRID,),
        in_specs=[
            pl.BlockSpec((_TC_ROWS, _TC_COLS), lambda i: (i, 0)),
            pl.BlockSpec((1, _TC_COLS), lambda i: (0, 0)),
        ],
        out_specs=pl.BlockSpec((1, 1), lambda i: (0, 0)),
        out_shape=jax.ShapeDtypeStruct((1, 1), jnp.float32),
    )(x2, coeff_tile)
    return out[0, 0]


def _f_pair(x):
    ax = jnp.abs(x)
    e = jnp.exp(-ax)
    s = e / (2.0 + e)
    s2 = s * s
    l1p = 2.0 * s * (1.0 + s2 * (1.0 / 3.0 + s2 * (0.2 + s2 * (1.0 / 7.0))))
    sp_pos = jnp.maximum(x, 0.0) + l1p
    sp_neg = sp_pos - x
    p = jnp.where(x >= 0, 1.0, e) / (1.0 + e)
    q1 = 1.0 - p
    return p * p * sp_pos, q1 * q1 * sp_neg


def _sc_body(cls_hbm, dir_hbm, rad_hbm, fw_hbm, src_hbm, lbl_hbm,
             tdir_hbm, trad_hbm, out_hbm,
             src_v, lbl_v, idx_v, idx3_v, cls_v, dir_v, rad_v, tdir_v, trad_v,
             fw_v, acc_v, sem):
    wid = lax.axis_index("s") * 2 + lax.axis_index("c")
    base = wid * N_PER_W

    pltpu.sync_copy(src_hbm.at[pl.ds(base, N_PER_W)], src_v)
    pltpu.sync_copy(lbl_hbm.at[pl.ds(base, N_PER_W)], lbl_v)
    pltpu.sync_copy(tdir_hbm.at[pl.ds(base, N_PER_W)], tdir_v)
    pltpu.sync_copy(trad_hbm.at[pl.ds(base, N_PER_W)], trad_v)
    pltpu.sync_copy(fw_hbm, fw_v)

    def _idx_step(j, _):
        s16 = src_v[pl.ds(j * 16, 16)]
        b = 2 * wid + jnp.where(j >= (T // 16), 1, 0)
        q16 = s16 + b * Q
        idx_v[pl.ds(j * 16, 16)] = q16
        q3 = q16 * 3
        idx3_v[pl.ds(j * 16, 16)] = q3
        idx3_v[pl.ds(N_PER_W + j * 16, 16)] = q3 + 1
        idx3_v[pl.ds(2 * N_PER_W + j * 16, 16)] = q3 + 2
        return 0

    lax.fori_loop(0, LG, _idx_step, 0)

    copies = []
    for c in range(N_CHUNKS):
        sl = pl.ds(c * IDX_CHUNK, IDX_CHUNK)
        copies.append(pltpu.async_copy(cls_hbm.at[idx_v.at[sl]],
                                       cls_v.at[sl], sem))
        copies.append(pltpu.async_copy(rad_hbm.at[idx_v.at[sl]],
                                       rad_v.at[sl], sem))
    for c in range(3 * N_CHUNKS):
        sl = pl.ds(c * IDX_CHUNK, IDX_CHUNK)
        copies.append(pltpu.async_copy(dir_hbm.at[idx3_v.at[sl]],
                                       dir_v.at[sl], sem))
    for cp in copies:
        cp.wait()

    k15 = jnp.full((16,), BG, jnp.int32)
    w15 = plsc.load_gather(fw_v, [k15])
    k0 = jnp.full((16,), 0, jnp.int32)
    k1 = jnp.full((16,), 1, jnp.int32)
    k2 = jnp.full((16,), 2, jnp.int32)
    lane = lax.iota(jnp.int32, 16)

    def _step(j, carry):
        a_corr, a_dir, a_rad = carry
        it16 = lane + j * 16
        lbl16 = lbl_v[pl.ds(j * 16, 16)]
        xl = plsc.load_gather(cls_v, [it16, lbl16])
        x15 = plsc.load_gather(cls_v, [it16, k15])
        wl = plsc.load_gather(fw_v, [lbl16])
        f0l, f1l = _f_pair(xl)
        f015, f115 = _f_pair(x15)
        a_corr = a_corr + wl * (0.25 * f1l - 0.75 * f0l) \
            + w15 * (0.75 * f015 - 0.25 * f115)
        for ki, kk in enumerate((k0, k1, k2)):
            dk = dir_v[pl.ds(ki * N_PER_W + j * 16, 16)]
            tk = plsc.load_gather(tdir_v, [it16, kk])
            a_dir = a_dir + jnp.abs(dk - tk)
        r16 = rad_v[pl.ds(j * 16, 16)]
        tr16 = trad_v[pl.ds(j * 16, 16)]
        a_rad = a_rad + jnp.abs(r16 - tr16)
        return a_corr, a_dir, a_rad

    z = jnp.zeros((16,), jnp.float32)
    a_corr, a_dir, a_rad = lax.fori_loop(0, LG, _step, (z, z, z))

    acc_v[pl.ds(0, 16)] = a_corr
    acc_v[pl.ds(16, 16)] = a_dir
    acc_v[pl.ds(32, 16)] = a_rad
    pltpu.sync_copy(acc_v, out_hbm.at[wid])


@functools.partial(jax.jit, static_argnames=())
def _sc_partials(cls2, dir2, rad1, fw, srcf, lblf, tdir2, tradf):
    mesh = plsc.VectorSubcoreMesh(core_axis_name="c", subcore_axis_name="s",
                                  num_cores=2, num_subcores=16)
    fn = pl.kernel(
        _sc_body,
        out_type=jax.ShapeDtypeStruct((NW, 48), jnp.float32),
        mesh=mesh,
        compiler_params=pltpu.CompilerParams(
            needs_layout_passes=False, use_tc_tiling_on_sc=False),
        scratch_types=[
            pltpu.VMEM((N_PER_W,), jnp.int32),
            pltpu.VMEM((N_PER_W,), jnp.int32),
            pltpu.VMEM((N_PER_W,), jnp.int32),
            pltpu.VMEM((3 * N_PER_W,), jnp.int32),
            pltpu.VMEM((N_PER_W, C), jnp.float32),
            pltpu.VMEM((3 * N_PER_W,), jnp.float32),
            pltpu.VMEM((N_PER_W,), jnp.float32),
            pltpu.VMEM((N_PER_W, 3), jnp.float32),
            pltpu.VMEM((N_PER_W,), jnp.float32),
            pltpu.VMEM((16,), jnp.float32),
            pltpu.VMEM((48,), jnp.float32),
            pltpu.SemaphoreType.DMA,
        ],
    )
    return fn(cls2, dir2, rad1, fw, srcf, lblf, tdir2, tradf)


def kernel(class_logits, direc_logits, radius_logits, focal_weights,
           target_directions, target_radii, src_idx, target_labels):
    fw = focal_weights.astype(jnp.float32)
    alpha_col = jnp.where(jnp.arange(C) == BG, 0.25, 0.75)
    coeff_tile = jnp.tile(fw * alpha_col, _TC_COLS // C)[None, :]

    s_bg = _bg_focal_sum(class_logits, coeff_tile)

    partials = _sc_partials(
        class_logits.reshape(B * Q, C),
        direc_logits.reshape(B * Q * 3),
        radius_logits.reshape(B * Q),
        fw,
        src_idx.reshape(N_ITEMS).astype(jnp.int32),
        target_labels.reshape(N_ITEMS).astype(jnp.int32),
        target_directions.reshape(N_ITEMS, 3),
        target_radii.reshape(N_ITEMS),
    )
    s_corr = jnp.sum(partials[:, :16])
    s_dir = jnp.sum(partials[:, 16:32])
    s_rad = jnp.sum(partials[:, 32:48])

    num_vessels = float(B * T)
    loss_class = (s_bg + s_corr) / (Q * num_vessels)
    return (W_CLASS * loss_class + W_DIR * s_dir / num_vessels
            + W_RAD * s_rad / num_vessels)


# speedup vs baseline: 7.4018x; 7.4018x over previous
import functools

import jax
import jax.numpy as jnp
from jax import lax
from jax.experimental import pallas as pl
from jax.experimental.pallas import tpu as pltpu
from jax.experimental.pallas import tpu_sc as plsc

B, Q, C, T = 64, 4096, 16, 512
BG = 15
W_CLASS, W_DIR, W_RAD = 1.0, 5.0, 2.0

N_ITEMS = B * T
NW = 32
N_PER_W = N_ITEMS // NW
LG = N_PER_W // 16
IDX_CHUNK = 128


_TC_ROWS = 4096
_TC_GRID = (B * Q * C) // (_TC_ROWS * 128)


def _bg_focal_body(x_ref, out_ref):
    i = pl.program_id(0)

    @pl.when(i == 0)
    def _init():
        out_ref[...] = jnp.zeros((512, 128), jnp.float32)

    x = x_ref[...]
    r = lax.broadcasted_iota(jnp.int32, (_TC_ROWS, 1), 0)
    is_bg = ((r >> 8) & 1 == 1) & ((r & 7) == 7)
    p = jax.nn.sigmoid(x)
    sp_pos = jnp.maximum(x, 0.0) + jnp.log1p(jnp.exp(-jnp.abs(x)))
    mod = jnp.where(is_bg, (1.0 - p) * (1.0 - p) * (sp_pos - x),
                    p * p * sp_pos)
    out_ref[...] += jnp.sum(mod.reshape(8, 512, 128), axis=0)


def _bg_focal_fold(class_native):
    return pl.pallas_call(
        _bg_focal_body,
        grid=(_TC_GRID,),
        in_specs=[pl.BlockSpec((_TC_ROWS, 128), lambda i: (i, 0))],
        out_specs=pl.BlockSpec((512, 128), lambda i: (0, 0)),
        out_shape=jax.ShapeDtypeStruct((512, 128), jnp.float32),
    )(class_native)


def _f_pair(x):
    ax = jnp.abs(x)
    e = jnp.exp(-ax)
    s = e / (2.0 + e)
    s2 = s * s
    l1p = 2.0 * s * (1.0 + s2 * (1.0 / 3.0 + s2 * (0.2 + s2 * (1.0 / 7.0))))
    sp_pos = jnp.maximum(x, 0.0) + l1p
    sp_neg = sp_pos - x
    p = jnp.where(x >= 0, 1.0, e) / (1.0 + e)
    q1 = 1.0 - p
    return p * p * sp_pos, q1 * q1 * sp_neg


def _sc_body(cls_hbm, dir_hbm, rad_hbm, fw_hbm, src_hbm, lbl_hbm,
             tdir_hbm, trad_hbm, out_hbm,
             src_v, lbl_v, icls_v, idir_v, irad_v,
             xl_v, x15_v, dirg_v, rad_v, tdir_v, trad_v,
             fw_v, acc_v, sem):
    wid = lax.axis_index("s") * 2 + lax.axis_index("c")
    base = wid * N_PER_W

    pltpu.sync_copy(src_hbm.at[pl.ds(base, N_PER_W)], src_v)
    pltpu.sync_copy(lbl_hbm.at[pl.ds(base, N_PER_W)], lbl_v)
    for c in range(3):
        pltpu.sync_copy(tdir_hbm.at[pl.ds(c * N_ITEMS + base, N_PER_W)],
                        tdir_v.at[pl.ds(c * N_PER_W, N_PER_W)])
    pltpu.sync_copy(trad_hbm.at[pl.ds(base, N_PER_W)], trad_v)
    pltpu.sync_copy(fw_hbm, fw_v)

    def _idx_step(j, _):
        q16 = src_v[pl.ds(j * 16, 16)]
        l16 = lbl_v[pl.ds(j * 16, 16)]
        b = 2 * wid + jnp.where(j >= (T // 16), 1, 0)
        qhi = (q16 >> 7) << 10
        qlo = q16 & 127
        cls_base = b * 65536 + qhi + qlo
        icls_v[pl.ds(j * 16, 16)] = cls_base + ((l16 >> 3) << 15) \
            + ((l16 & 7) << 7)
        icls_v[pl.ds(N_PER_W + j * 16, 16)] = cls_base + 32768 + 7 * 128
        dir_base = ((b >> 3) << 15) + ((b & 7) << 7) + qhi + qlo
        idir_v[pl.ds(j * 16, 16)] = dir_base
        idir_v[pl.ds(N_PER_W + j * 16, 16)] = dir_base + 262144
        idir_v[pl.ds(2 * N_PER_W + j * 16, 16)] = dir_base + 2 * 262144
        irad_v[pl.ds(j * 16, 16)] = b * 4096 + q16
        return 0

    lax.fori_loop(0, LG, _idx_step, 0)

    copies = []

    def _gather(tab, iv, off, dv, n):
        for c in range(n // IDX_CHUNK):
            si = pl.ds(off + c * IDX_CHUNK, IDX_CHUNK)
            sd = pl.ds(c * IDX_CHUNK, IDX_CHUNK)
            copies.append(pltpu.async_copy(tab.at[iv.at[si]], dv.at[sd], sem))

    _gather(cls_hbm, icls_v, 0, xl_v, N_PER_W)
    _gather(cls_hbm, icls_v, N_PER_W, x15_v, N_PER_W)
    _gather(dir_hbm, idir_v, 0, dirg_v, 3 * N_PER_W)
    _gather(rad_hbm, irad_v, 0, rad_v, N_PER_W)
    for cp in copies:
        cp.wait()

    k15 = jnp.full((16,), BG, jnp.int32)
    w15 = plsc.load_gather(fw_v, [k15])

    def _step(j, carry):
        a_corr, a_dir, a_rad = carry
        lbl16 = lbl_v[pl.ds(j * 16, 16)]
        xl = xl_v[pl.ds(j * 16, 16)]
        x15 = x15_v[pl.ds(j * 16, 16)]
        wl = plsc.load_gather(fw_v, [lbl16])
        f0l, f1l = _f_pair(xl)
        f015, f115 = _f_pair(x15)
        a_corr = a_corr + wl * (0.25 * f1l - 0.75 * f0l) \
            + w15 * (0.75 * f015 - 0.25 * f115)
        for c in range(3):
            dk = dirg_v[pl.ds(c * N_PER_W + j * 16, 16)]
            tk = tdir_v[pl.ds(c * N_PER_W + j * 16, 16)]
            a_dir = a_dir + jnp.abs(dk - tk)
        r16 = rad_v[pl.ds(j * 16, 16)]
        tr16 = trad_v[pl.ds(j * 16, 16)]
        a_rad = a_rad + jnp.abs(r16 - tr16)
        return a_corr, a_dir, a_rad

    z = jnp.zeros((16,), jnp.float32)
    a_corr, a_dir, a_rad = lax.fori_loop(0, LG, _step, (z, z, z))

    acc_v[pl.ds(0, 16)] = a_corr
    acc_v[pl.ds(16, 16)] = a_dir
    acc_v[pl.ds(32, 16)] = a_rad
    pltpu.sync_copy(acc_v, out_hbm.at[wid])


def _sc_partials(cls_flat, dir_flat, rad_flat, fw, srcf, lblf, tdirf, tradf):
    mesh = plsc.VectorSubcoreMesh(core_axis_name="c", subcore_axis_name="s",
                                  num_cores=2, num_subcores=16)
    fn = pl.kernel(
        _sc_body,
        out_type=jax.ShapeDtypeStruct((NW, 48), jnp.float32),
        mesh=mesh,
        compiler_params=pltpu.CompilerParams(
            needs_layout_passes=False, use_tc_tiling_on_sc=False),
        scratch_types=[
            pltpu.VMEM((N_PER_W,), jnp.int32),
            pltpu.VMEM((N_PER_W,), jnp.int32),
            pltpu.VMEM((2 * N_PER_W,), jnp.int32),
            pltpu.VMEM((3 * N_PER_W,), jnp.int32),
            pltpu.VMEM((N_PER_W,), jnp.int32),
            pltpu.VMEM((N_PER_W,), jnp.float32),
            pltpu.VMEM((N_PER_W,), jnp.float32),
            pltpu.VMEM((3 * N_PER_W,), jnp.float32),
            pltpu.VMEM((N_PER_W,), jnp.float32),
            pltpu.VMEM((3 * N_PER_W,), jnp.float32),
            pltpu.VMEM((N_PER_W,), jnp.float32),
            pltpu.VMEM((16,), jnp.float32),
            pltpu.VMEM((48,), jnp.float32),
            pltpu.SemaphoreType.DMA,
        ],
    )
    return fn(cls_flat, dir_flat, rad_flat, fw, srcf, lblf, tdirf, tradf)


def kernel(class_logits, direc_logits, radius_logits, focal_weights,
           target_directions, target_radii, src_idx, target_labels):
    fw = focal_weights.astype(jnp.float32)

    cls_native = (class_logits.reshape(B, 32, 128, 2, 8)
                  .transpose(0, 3, 1, 4, 2).reshape(B * Q * C // 128, 128))
    dir_native = (direc_logits.reshape(8, 8, 32, 128, 3)
                  .transpose(4, 0, 2, 1, 3).reshape(B * Q * 3))
    rad_native = radius_logits.reshape(B * Q)

    fold = _bg_focal_fold(cls_native)

    partials = _sc_partials(
        cls_native.reshape(B * Q * C),
        dir_native,
        rad_native,
        fw,
        src_idx.reshape(N_ITEMS).astype(jnp.int32),
        target_labels.reshape(N_ITEMS).astype(jnp.int32),
        target_directions.transpose(2, 0, 1).reshape(3 * N_ITEMS),
        target_radii.reshape(N_ITEMS),
    )
    s_corr = jnp.sum(partials[:, :16])
    s_dir = jnp.sum(partials[:, 16:32])
    s_rad = jnp.sum(partials[:, 32:48])

    m = jnp.arange(512)
    ch = 8 * ((m >> 8) & 1) + (m & 7)
    coeff = fw[ch] * jnp.where(ch == BG, 0.25, 0.75)
    s_bg = jnp.dot(coeff, jnp.sum(fold, axis=1))

    num_vessels = float(B * T)
    loss_class = (s_bg + s_corr) / (Q * num_vessels)
    return (W_CLASS * loss_class + W_DIR * s_dir / num_vessels
            + W_RAD * s_rad / num_vessels)


# speedup vs baseline: 10.1328x; 1.3690x over previous
import functools

import jax
import jax.numpy as jnp
from jax import lax
from jax.experimental import pallas as pl
from jax.experimental.pallas import tpu as pltpu
from jax.experimental.pallas import tpu_sc as plsc

B, Q, C, T = 64, 4096, 16, 512
BG = 15
W_CLASS, W_DIR, W_RAD = 1.0, 5.0, 2.0

N_ITEMS = B * T
NW = 32
N_PER_W = N_ITEMS // NW
LG = N_PER_W // 16
IDX_CHUNK = 128


_TC_ROWS = 4096
_TC_GRID = (B * Q * C) // (_TC_ROWS * 128)


def _bg_focal_body(x_ref, out_ref):
    i = pl.program_id(0)

    @pl.when(i == 0)
    def _init():
        out_ref[...] = jnp.zeros((512, 128), jnp.float32)

    x = x_ref[...]
    r = lax.broadcasted_iota(jnp.int32, (_TC_ROWS, 1), 0)
    is_bg = ((r >> 8) & 1 == 1) & ((r & 7) == 7)
    p = jax.nn.sigmoid(x)
    sp_pos = jnp.maximum(x, 0.0) + jnp.log1p(jnp.exp(-jnp.abs(x)))
    mod = jnp.where(is_bg, (1.0 - p) * (1.0 - p) * (sp_pos - x),
                    p * p * sp_pos)
    out_ref[...] += jnp.sum(mod.reshape(8, 512, 128), axis=0)


def _bg_focal_fold(class_native):
    return pl.pallas_call(
        _bg_focal_body,
        grid=(_TC_GRID,),
        in_specs=[pl.BlockSpec((_TC_ROWS, 128), lambda i: (i, 0))],
        out_specs=pl.BlockSpec((512, 128), lambda i: (0, 0)),
        out_shape=jax.ShapeDtypeStruct((512, 128), jnp.float32),
    )(class_native)


def _f_pair(x):
    ax = jnp.abs(x)
    e = jnp.exp(-ax)
    s = e / (2.0 + e)
    s2 = s * s
    l1p = 2.0 * s * (1.0 + s2 * (1.0 / 3.0 + s2 * (0.2 + s2 * (1.0 / 7.0))))
    sp_pos = jnp.maximum(x, 0.0) + l1p
    sp_neg = sp_pos - x
    p = jnp.where(x >= 0, 1.0, e) / (1.0 + e)
    q1 = 1.0 - p
    return p * p * sp_pos, q1 * q1 * sp_neg


def _sc_body(cls_hbm, dir_hbm, rad_hbm, fw_hbm, src_hbm, lbl_hbm,
             tdir_hbm, trad_hbm, out_hbm,
             src_v, lbl_v, icls_v, idir_v, irad_v,
             xl_v, x15_v, dirg_v, rad_v, tdir_v, trad_v,
             fw_v, acc_v, sem):
    wid = lax.axis_index("s") * 2 + lax.axis_index("c")

    bt = wid // 4
    bs0 = (2 * wid) % 8
    stage = []
    for i01 in range(2):
        for tt in range(4):
            src_off = bt * 4096 + tt * 1024 + (bs0 + i01) * 128
            dst = pl.ds(i01 * 512 + tt * 128, 128)
            run = pl.ds(src_off, 128)
            stage.append(pltpu.async_copy(src_hbm.at[run], src_v.at[dst], sem))
            stage.append(pltpu.async_copy(lbl_hbm.at[run], lbl_v.at[dst], sem))
            stage.append(pltpu.async_copy(trad_hbm.at[run], trad_v.at[dst],
                                          sem))
            for c in range(3):
                stage.append(pltpu.async_copy(
                    tdir_hbm.at[pl.ds(c * N_ITEMS + src_off, 128)],
                    tdir_v.at[pl.ds(c * N_PER_W + i01 * 512 + tt * 128, 128)],
                    sem))
    pltpu.sync_copy(fw_hbm, fw_v)
    for cp in stage:
        cp.wait()

    def _idx_step(j, _):
        q16 = src_v[pl.ds(j * 16, 16)]
        l16 = lbl_v[pl.ds(j * 16, 16)]
        b = 2 * wid + jnp.where(j >= (T // 16), 1, 0)
        qhi = (q16 >> 7) << 10
        qlo = q16 & 127
        cls_base = b * 65536 + qhi + qlo
        icls_v[pl.ds(j * 16, 16)] = cls_base + ((l16 >> 3) << 15) \
            + ((l16 & 7) << 7)
        icls_v[pl.ds(N_PER_W + j * 16, 16)] = cls_base + 32768 + 7 * 128
        dir_base = ((b >> 3) << 15) + ((b & 7) << 7) + qhi + qlo
        idir_v[pl.ds(j * 16, 16)] = dir_base
        idir_v[pl.ds(N_PER_W + j * 16, 16)] = dir_base + 262144
        idir_v[pl.ds(2 * N_PER_W + j * 16, 16)] = dir_base + 2 * 262144
        irad_v[pl.ds(j * 16, 16)] = b * 4096 + q16
        return 0

    lax.fori_loop(0, LG, _idx_step, 0)

    copies = []

    def _gather(tab, iv, off, dv, n):
        for c in range(n // IDX_CHUNK):
            si = pl.ds(off + c * IDX_CHUNK, IDX_CHUNK)
            sd = pl.ds(c * IDX_CHUNK, IDX_CHUNK)
            copies.append(pltpu.async_copy(tab.at[iv.at[si]], dv.at[sd], sem))

    _gather(cls_hbm, icls_v, 0, xl_v, N_PER_W)
    _gather(cls_hbm, icls_v, N_PER_W, x15_v, N_PER_W)
    _gather(dir_hbm, idir_v, 0, dirg_v, 3 * N_PER_W)
    _gather(rad_hbm, irad_v, 0, rad_v, N_PER_W)
    for cp in copies:
        cp.wait()

    k15 = jnp.full((16,), BG, jnp.int32)
    w15 = plsc.load_gather(fw_v, [k15])

    def _step(j, carry):
        a_corr, a_dir, a_rad = carry
        lbl16 = lbl_v[pl.ds(j * 16, 16)]
        xl = xl_v[pl.ds(j * 16, 16)]
        x15 = x15_v[pl.ds(j * 16, 16)]
        wl = plsc.load_gather(fw_v, [lbl16])
        f0l, f1l = _f_pair(xl)
        f015, f115 = _f_pair(x15)
        a_corr = a_corr + wl * (0.25 * f1l - 0.75 * f0l) \
            + w15 * (0.75 * f015 - 0.25 * f115)
        for c in range(3):
            dk = dirg_v[pl.ds(c * N_PER_W + j * 16, 16)]
            tk = tdir_v[pl.ds(c * N_PER_W + j * 16, 16)]
            a_dir = a_dir + jnp.abs(dk - tk)
        r16 = rad_v[pl.ds(j * 16, 16)]
        tr16 = trad_v[pl.ds(j * 16, 16)]
        a_rad = a_rad + jnp.abs(r16 - tr16)
        return a_corr, a_dir, a_rad

    z = jnp.zeros((16,), jnp.float32)
    a_corr, a_dir, a_rad = lax.fori_loop(0, LG, _step, (z, z, z))

    acc_v[pl.ds(0, 16)] = a_corr
    acc_v[pl.ds(16, 16)] = a_dir
    acc_v[pl.ds(32, 16)] = a_rad
    pltpu.sync_copy(acc_v, out_hbm.at[wid])


def _sc_partials(cls_flat, dir_flat, rad_flat, fw, srcf, lblf, tdirf, tradf):
    mesh = plsc.VectorSubcoreMesh(core_axis_name="c", subcore_axis_name="s",
                                  num_cores=2, num_subcores=16)
    fn = pl.kernel(
        _sc_body,
        out_type=jax.ShapeDtypeStruct((NW, 48), jnp.float32),
        mesh=mesh,
        compiler_params=pltpu.CompilerParams(
            needs_layout_passes=False, use_tc_tiling_on_sc=False),
        scratch_types=[
            pltpu.VMEM((N_PER_W,), jnp.int32),
            pltpu.VMEM((N_PER_W,), jnp.int32),
            pltpu.VMEM((2 * N_PER_W,), jnp.int32),
            pltpu.VMEM((3 * N_PER_W,), jnp.int32),
            pltpu.VMEM((N_PER_W,), jnp.int32),
            pltpu.VMEM((N_PER_W,), jnp.float32),
            pltpu.VMEM((N_PER_W,), jnp.float32),
            pltpu.VMEM((3 * N_PER_W,), jnp.float32),
            pltpu.VMEM((N_PER_W,), jnp.float32),
            pltpu.VMEM((3 * N_PER_W,), jnp.float32),
            pltpu.VMEM((N_PER_W,), jnp.float32),
            pltpu.VMEM((16,), jnp.float32),
            pltpu.VMEM((48,), jnp.float32),
            pltpu.SemaphoreType.DMA,
        ],
    )
    return fn(cls_flat, dir_flat, rad_flat, fw, srcf, lblf, tdirf, tradf)


def kernel(class_logits, direc_logits, radius_logits, focal_weights,
           target_directions, target_radii, src_idx, target_labels):
    fw = focal_weights.astype(jnp.float32)

    cls_native = (class_logits.reshape(B, 32, 128, 2, 8)
                  .transpose(0, 3, 1, 4, 2).reshape(B * Q * C // 128, 128))
    dir_native = (direc_logits.reshape(8, 8, 32, 128, 3)
                  .transpose(4, 0, 2, 1, 3).reshape(B * Q * 3))
    rad_native = radius_logits.reshape(B * Q)

    fold = _bg_focal_fold(cls_native)

    def _item_native(a):
        return (a.reshape(8, 8, 4, 128).transpose(0, 2, 1, 3)
                .reshape(N_ITEMS))

    partials = _sc_partials(
        cls_native.reshape(B * Q * C),
        dir_native,
        rad_native,
        fw,
        _item_native(src_idx.astype(jnp.int32)),
        _item_native(target_labels.astype(jnp.int32)),
        (target_directions.reshape(8, 8, 4, 128, 3)
         .transpose(4, 0, 2, 1, 3).reshape(3 * N_ITEMS)),
        _item_native(target_radii),
    )
    s_corr = jnp.sum(partials[:, :16])
    s_dir = jnp.sum(partials[:, 16:32])
    s_rad = jnp.sum(partials[:, 32:48])

    alpha2 = jnp.full((2, 8), 0.75).at[1, 7].set(0.25)
    w2 = fw.reshape(2, 8) * alpha2
    coeff = jnp.broadcast_to(w2[:, None, :], (2, 32, 8)).reshape(512)
    s_bg = jnp.sum(fold * coeff[:, None])

    num_vessels = float(B * T)
    loss_class = (s_bg + s_corr) / (Q * num_vessels)
    return (W_CLASS * loss_class + W_DIR * s_dir / num_vessels
            + W_RAD * s_rad / num_vessels)


# speedup vs baseline: 10.9770x; 1.0833x over previous
import functools

import jax
import jax.numpy as jnp
from jax import lax
from jax.experimental import pallas as pl
from jax.experimental.pallas import tpu as pltpu
from jax.experimental.pallas import tpu_sc as plsc

B, Q, C, T = 64, 4096, 16, 512
BG = 15
W_CLASS, W_DIR, W_RAD = 1.0, 5.0, 2.0

N_ITEMS = B * T
NW = 32
N_PER_W = N_ITEMS // NW
LG = N_PER_W // 16
IDX_CHUNK = 128


_TC_ROWS = 4096
_TC_GRID = (B * Q * C) // (_TC_ROWS * 128)


def _bg_focal_body(x_ref, out_ref):
    i = pl.program_id(0)

    @pl.when(i == 0)
    def _init():
        out_ref[...] = jnp.zeros((1024, 128), jnp.float32)

    x = x_ref[...]
    ax = jnp.abs(x)
    e = jnp.exp(-ax)
    u = 1.0 + e
    r = 1.0 / u
    p = jnp.where(x >= 0, r, e * r)
    sp = jnp.maximum(x, 0.0) + jnp.log(u)
    t0 = p * p * sp
    q1 = 1.0 - p
    t1 = q1 * q1 * (sp - x)
    out_ref[pl.ds(0, 512), :] += jnp.sum(t0.reshape(8, 512, 128), axis=0)
    out_ref[pl.ds(512, 512), :] += jnp.sum(t1.reshape(8, 512, 128), axis=0)


def _bg_focal_fold(class_native):
    return pl.pallas_call(
        _bg_focal_body,
        grid=(_TC_GRID,),
        in_specs=[pl.BlockSpec((_TC_ROWS, 128), lambda i: (i, 0))],
        out_specs=pl.BlockSpec((1024, 128), lambda i: (0, 0)),
        out_shape=jax.ShapeDtypeStruct((1024, 128), jnp.float32),
    )(class_native)


def _f_pair(x):
    ax = jnp.abs(x)
    e = jnp.exp(-ax)
    s = e / (2.0 + e)
    s2 = s * s
    l1p = 2.0 * s * (1.0 + s2 * (1.0 / 3.0 + s2 * (0.2 + s2 * (1.0 / 7.0))))
    sp_pos = jnp.maximum(x, 0.0) + l1p
    sp_neg = sp_pos - x
    p = jnp.where(x >= 0, 1.0, e) / (1.0 + e)
    q1 = 1.0 - p
    return p * p * sp_pos, q1 * q1 * sp_neg


def _sc_body(cls_hbm, dir_hbm, rad_hbm, fw_hbm, src_hbm, lbl_hbm,
             tdir_hbm, trad_hbm, out_hbm,
             src_v, lbl_v, icls_v, idir_v, irad_v,
             xl_v, x15_v, dirg_v, rad_v, tdir_v, trad_v,
             fw_v, acc_v, sem):
    wid = lax.axis_index("s") * 2 + lax.axis_index("c")

    bt = wid // 4
    bs0 = (2 * wid) % 8
    stage = []
    for i01 in range(2):
        for tt in range(4):
            src_off = bt * 4096 + tt * 1024 + (bs0 + i01) * 128
            dst = pl.ds(i01 * 512 + tt * 128, 128)
            run = pl.ds(src_off, 128)
            stage.append(pltpu.async_copy(src_hbm.at[run], src_v.at[dst], sem))
            stage.append(pltpu.async_copy(lbl_hbm.at[run], lbl_v.at[dst], sem))
            stage.append(pltpu.async_copy(trad_hbm.at[run], trad_v.at[dst],
                                          sem))
            for c in range(3):
                stage.append(pltpu.async_copy(
                    tdir_hbm.at[pl.ds(c * N_ITEMS + src_off, 128)],
                    tdir_v.at[pl.ds(c * N_PER_W + i01 * 512 + tt * 128, 128)],
                    sem))
    pltpu.sync_copy(fw_hbm, fw_v)
    for cp in stage:
        cp.wait()

    def _idx_step(j, _):
        q16 = src_v[pl.ds(j * 16, 16)]
        l16 = lbl_v[pl.ds(j * 16, 16)]
        b = 2 * wid + jnp.where(j >= (T // 16), 1, 0)
        qhi = (q16 >> 7) << 10
        qlo = q16 & 127
        cls_base = b * 65536 + qhi + qlo
        icls_v[pl.ds(j * 16, 16)] = cls_base + ((l16 >> 3) << 15) \
            + ((l16 & 7) << 7)
        icls_v[pl.ds(N_PER_W + j * 16, 16)] = cls_base + 32768 + 7 * 128
        dir_base = ((b >> 3) << 15) + ((b & 7) << 7) + qhi + qlo
        idir_v[pl.ds(j * 16, 16)] = dir_base
        idir_v[pl.ds(N_PER_W + j * 16, 16)] = dir_base + 262144
        idir_v[pl.ds(2 * N_PER_W + j * 16, 16)] = dir_base + 2 * 262144
        irad_v[pl.ds(j * 16, 16)] = b * 4096 + q16
        return 0

    lax.fori_loop(0, LG, _idx_step, 0)

    copies = []

    def _gather(tab, iv, off, dv, n):
        for c in range(n // IDX_CHUNK):
            si = pl.ds(off + c * IDX_CHUNK, IDX_CHUNK)
            sd = pl.ds(c * IDX_CHUNK, IDX_CHUNK)
            copies.append(pltpu.async_copy(tab.at[iv.at[si]], dv.at[sd], sem))

    _gather(cls_hbm, icls_v, 0, xl_v, N_PER_W)
    _gather(cls_hbm, icls_v, N_PER_W, x15_v, N_PER_W)
    _gather(dir_hbm, idir_v, 0, dirg_v, 3 * N_PER_W)
    _gather(rad_hbm, irad_v, 0, rad_v, N_PER_W)
    for cp in copies:
        cp.wait()

    k15 = jnp.full((16,), BG, jnp.int32)
    w15 = plsc.load_gather(fw_v, [k15])

    def _step(j, carry):
        a_corr, a_dir, a_rad = carry
        lbl16 = lbl_v[pl.ds(j * 16, 16)]
        xl = xl_v[pl.ds(j * 16, 16)]
        x15 = x15_v[pl.ds(j * 16, 16)]
        wl = plsc.load_gather(fw_v, [lbl16])
        f0l, f1l = _f_pair(xl)
        f015, f115 = _f_pair(x15)
        a_corr = a_corr + wl * (0.25 * f1l - 0.75 * f0l) \
            + w15 * (0.75 * f015 - 0.25 * f115)
        for c in range(3):
            dk = dirg_v[pl.ds(c * N_PER_W + j * 16, 16)]
            tk = tdir_v[pl.ds(c * N_PER_W + j * 16, 16)]
            a_dir = a_dir + jnp.abs(dk - tk)
        r16 = rad_v[pl.ds(j * 16, 16)]
        tr16 = trad_v[pl.ds(j * 16, 16)]
        a_rad = a_rad + jnp.abs(r16 - tr16)
        return a_corr, a_dir, a_rad

    z = jnp.zeros((16,), jnp.float32)
    a_corr, a_dir, a_rad = lax.fori_loop(0, LG, _step, (z, z, z))

    acc_v[pl.ds(0, 16)] = a_corr
    acc_v[pl.ds(16, 16)] = a_dir
    acc_v[pl.ds(32, 16)] = a_rad
    pltpu.sync_copy(acc_v, out_hbm.at[wid])


def _sc_partials(cls_flat, dir_flat, rad_flat, fw, srcf, lblf, tdirf, tradf):
    mesh = plsc.VectorSubcoreMesh(core_axis_name="c", subcore_axis_name="s",
                                  num_cores=2, num_subcores=16)
    fn = pl.kernel(
        _sc_body,
        out_type=jax.ShapeDtypeStruct((NW, 48), jnp.float32),
        mesh=mesh,
        compiler_params=pltpu.CompilerParams(
            needs_layout_passes=False, use_tc_tiling_on_sc=False),
        scratch_types=[
            pltpu.VMEM((N_PER_W,), jnp.int32),
            pltpu.VMEM((N_PER_W,), jnp.int32),
            pltpu.VMEM((2 * N_PER_W,), jnp.int32),
            pltpu.VMEM((3 * N_PER_W,), jnp.int32),
            pltpu.VMEM((N_PER_W,), jnp.int32),
            pltpu.VMEM((N_PER_W,), jnp.float32),
            pltpu.VMEM((N_PER_W,), jnp.float32),
            pltpu.VMEM((3 * N_PER_W,), jnp.float32),
            pltpu.VMEM((N_PER_W,), jnp.float32),
            pltpu.VMEM((3 * N_PER_W,), jnp.float32),
            pltpu.VMEM((N_PER_W,), jnp.float32),
            pltpu.VMEM((16,), jnp.float32),
            pltpu.VMEM((48,), jnp.float32),
            pltpu.SemaphoreType.DMA,
        ],
    )
    return fn(cls_flat, dir_flat, rad_flat, fw, srcf, lblf, tdirf, tradf)


def kernel(class_logits, direc_logits, radius_logits, focal_weights,
           target_directions, target_radii, src_idx, target_labels):
    fw = focal_weights.astype(jnp.float32)

    cls_native = (class_logits.reshape(B, 32, 128, 2, 8)
                  .transpose(0, 3, 1, 4, 2).reshape(B * Q * C // 128, 128))
    dir_native = (direc_logits.reshape(8, 8, 32, 128, 3)
                  .transpose(4, 0, 2, 1, 3).reshape(B * Q * 3))
    rad_native = radius_logits.reshape(B * Q)

    fold = _bg_focal_fold(cls_native)

    def _item_native(a):
        return (a.reshape(8, 8, 4, 128).transpose(0, 2, 1, 3)
                .reshape(N_ITEMS))

    partials = _sc_partials(
        cls_native.reshape(B * Q * C),
        dir_native,
        rad_native,
        fw,
        _item_native(src_idx.astype(jnp.int32)),
        _item_native(target_labels.astype(jnp.int32)),
        (target_directions.reshape(8, 8, 4, 128, 3)
         .transpose(4, 0, 2, 1, 3).reshape(3 * N_ITEMS)),
        _item_native(target_radii),
    )
    s_corr = jnp.sum(partials[:, :16])
    s_dir = jnp.sum(partials[:, 16:32])
    s_rad = jnp.sum(partials[:, 32:48])

    w2 = fw.reshape(2, 8)
    c0 = 0.75 * w2.at[1, 7].set(0.0)
    c1 = jnp.zeros((2, 8)).at[1, 7].set(0.25 * fw[BG])
    coeff = jnp.concatenate([
        jnp.broadcast_to(c0[:, None, :], (2, 32, 8)).reshape(512),
        jnp.broadcast_to(c1[:, None, :], (2, 32, 8)).reshape(512)])
    s_bg = jnp.sum(fold * coeff[:, None])

    num_vessels = float(B * T)
    loss_class = (s_bg + s_corr) / (Q * num_vessels)
    return (W_CLASS * loss_class + W_DIR * s_dir / num_vessels
            + W_RAD * s_rad / num_vessels)
